# Initial kernel scaffold; baseline (speedup 1.0000x reference)
import jax
import jax.numpy as jnp
from jax.experimental import pallas as pl


def kernel(state, edge_index, Wc, bc, W1, b1, W2, b2, W3, b3, deterministic):
    raise NotImplementedError("write your pallas kernel here")


# speedup vs baseline: 25.9465x; 25.9465x over previous
import functools

import jax
import jax.numpy as jnp
from jax import lax
from jax.experimental import pallas as pl
from jax.experimental.pallas import tpu as pltpu
from jax.experimental.pallas import tpu_sc as plsc

N = 10000
E = 320000
D = 128
H = 32
A = 8

NC = 2
NS = 16
NW = NC * NS
L = 16

EPT = E // NW
CHUNK = 125
NCHUNK = EPT // CHUNK
NP = 10240
RPT = NP // NS
ZCH = 128
NZ = RPT // ZCH

ROWS_TC = 1000
GRID_TC = N // ROWS_TC

_mesh = plsc.VectorSubcoreMesh(core_axis_name="c", subcore_axis_name="s")


def _deg_body(dst_hbm, out_hbm, dst_v, ones_v, zv, acc_sh):
    c = lax.axis_index("c")
    s = lax.axis_index("s")
    wid = c * NS + s

    zeros16 = jnp.zeros((L,), jnp.float32)
    ones16 = jnp.ones((L,), jnp.float32)

    pltpu.sync_copy(dst_hbm.at[wid], dst_v)

    def _z(r, _):
        zv[r] = zeros16
        return 0
    lax.fori_loop(0, ZCH, _z, 0)
    for t in range(NZ):
        pltpu.sync_copy(zv, acc_sh.at[pl.ds(s * RPT + t * ZCH, ZCH)])

    def _o(r, _):
        ones_v[r] = ones16
        return 0
    lax.fori_loop(0, CHUNK, _o, 0)

    plsc.subcore_barrier()

    def _body(j, _):
        pltpu.sync_copy(ones_v, acc_sh.at[dst_v.at[j]], add=True)
        return 0
    lax.fori_loop(0, NCHUNK, _body, 0)

    plsc.subcore_barrier()

    for t in range(NZ):
        pltpu.sync_copy(acc_sh.at[pl.ds(s * RPT + t * ZCH, ZCH)], zv)
        pltpu.sync_copy(zv, out_hbm.at[c, pl.ds(s * RPT + t * ZCH, ZCH)])


_deg_kernel = pl.kernel(
    _deg_body,
    out_type=jax.ShapeDtypeStruct((NC, NP, L), jnp.float32),
    mesh=_mesh,
    scratch_types=[
        pltpu.VMEM((NCHUNK, CHUNK), jnp.int32),
        pltpu.VMEM((CHUNK, L), jnp.float32),
        pltpu.VMEM((ZCH, L), jnp.float32),
        pltpu.VMEM_SHARED((NP, L), jnp.float32),
    ],
    compiler_params=pltpu.CompilerParams(use_tc_tiling_on_sc=False),
)


def _agg_body(g_hbm, src_hbm, dst_hbm, out_hbm, src_v, dst_v, rows_v,
              acc_sh, sem):
    c = lax.axis_index("c")
    s = lax.axis_index("s")
    wid = c * NS + s

    zeros16 = jnp.zeros((L,), jnp.float32)

    pltpu.sync_copy(src_hbm.at[wid], src_v)
    pltpu.sync_copy(dst_hbm.at[wid], dst_v)

    def _z(r, _):
        for k in range(D // L):
            rows_v[r, pl.ds(k * L, L)] = zeros16
        return 0
    lax.fori_loop(0, ZCH, _z, 0)
    for t in range(NZ):
        pltpu.sync_copy(rows_v, acc_sh.at[pl.ds(s * RPT + t * ZCH, ZCH)])

    plsc.subcore_barrier()

    def _body(j, _):
        pltpu.async_copy(g_hbm.at[src_v.at[j]], rows_v.at[pl.ds(0, CHUNK)],
                         sem).wait()
        pltpu.sync_copy(rows_v.at[pl.ds(0, CHUNK)],
                        acc_sh.at[dst_v.at[j]], add=True)
        return 0
    lax.fori_loop(0, NCHUNK, _body, 0)

    plsc.subcore_barrier()

    for t in range(NZ):
        pltpu.sync_copy(acc_sh.at[pl.ds(s * RPT + t * ZCH, ZCH)], rows_v)
        pltpu.sync_copy(rows_v, out_hbm.at[c, pl.ds(s * RPT + t * ZCH, ZCH)])


_agg_kernel = pl.kernel(
    _agg_body,
    out_type=jax.ShapeDtypeStruct((NC, NP, D), jnp.float32),
    mesh=_mesh,
    scratch_types=[
        pltpu.VMEM((NCHUNK, CHUNK), jnp.int32),
        pltpu.VMEM((NCHUNK, CHUNK), jnp.int32),
        pltpu.VMEM((ZCH, D), jnp.float32),
        pltpu.VMEM_SHARED((NP, D), jnp.float32),
        pltpu.SemaphoreType.DMA,
    ],
    compiler_params=pltpu.CompilerParams(use_tc_tiling_on_sc=False),
)


def _scale_body(state_ref, wc_ref, degp_ref, g_ref):
    deg = degp_ref[0, :, 0] + degp_ref[1, :, 0] + 1.0
    dinv = lax.rsqrt(deg)
    h = jnp.dot(state_ref[...], wc_ref[...], preferred_element_type=jnp.float32)
    g_ref[...] = h * dinv[:, None]


def _scale_call(state, Wc, degp):
    return pl.pallas_call(
        _scale_body,
        grid=(GRID_TC,),
        in_specs=[
            pl.BlockSpec((ROWS_TC, D), lambda i: (i, 0)),
            pl.BlockSpec((D, D), lambda i: (0, 0)),
            pl.BlockSpec((NC, ROWS_TC, L), lambda i: (0, i, 0)),
        ],
        out_specs=pl.BlockSpec((ROWS_TC, D), lambda i: (i, 0)),
        out_shape=jax.ShapeDtypeStruct((N, D), jnp.float32),
    )(state, Wc, degp)


def _head_body(p_ref, g_ref, state_ref, degp_ref, bc_ref, w1_ref, b1_ref,
               w2_ref, b2_ref, w3_ref, b3_ref, conc_ref):
    deg = degp_ref[0, :, 0] + degp_ref[1, :, 0] + 1.0
    dinv = lax.rsqrt(deg)
    agg = p_ref[0] + p_ref[1] + g_ref[...]
    gcn = jnp.maximum(agg * dinv[:, None] + bc_ref[...], 0.0)
    x = gcn + state_ref[...]
    t = jnp.dot(x, w1_ref[...], preferred_element_type=jnp.float32) + b1_ref[...]
    t = jnp.where(t > 0, t, 0.01 * t)
    t = jnp.dot(t, w2_ref[...], preferred_element_type=jnp.float32) + b2_ref[...]
    t = jnp.where(t > 0, t, 0.01 * t)
    t = jnp.sum(t * w3_ref[...], axis=1, keepdims=True) + b3_ref[...]
    conc_ref[...] = jnp.maximum(t, 0.0) + jnp.log1p(jnp.exp(-jnp.abs(t)))


def _head_call(parts, g, state, degp, bc, W1, b1, W2, b2, W3r, b3):
    return pl.pallas_call(
        _head_body,
        grid=(GRID_TC,),
        in_specs=[
            pl.BlockSpec((NC, ROWS_TC, D), lambda i: (0, i, 0)),
            pl.BlockSpec((ROWS_TC, D), lambda i: (i, 0)),
            pl.BlockSpec((ROWS_TC, D), lambda i: (i, 0)),
            pl.BlockSpec((NC, ROWS_TC, L), lambda i: (0, i, 0)),
            pl.BlockSpec((1, D), lambda i: (0, 0)),
            pl.BlockSpec((D, H), lambda i: (0, 0)),
            pl.BlockSpec((1, H), lambda i: (0, 0)),
            pl.BlockSpec((H, H), lambda i: (0, 0)),
            pl.BlockSpec((1, H), lambda i: (0, 0)),
            pl.BlockSpec((1, H), lambda i: (0, 0)),
            pl.BlockSpec((1, 1), lambda i: (0, 0)),
        ],
        out_specs=pl.BlockSpec((ROWS_TC, 1), lambda i: (i, 0)),
        out_shape=jax.ShapeDtypeStruct((N, 1), jnp.float32),
    )(parts, g, state, degp, bc, W1, b1, W2, b2, W3r, b3)


def _norm_body(conc_ref, out_ref):
    cv = conc_ref[...]
    out_ref[...] = cv / (jnp.sum(cv) + 1e-20)


def _norm_call(conc16):
    return pl.pallas_call(
        _norm_body,
        out_shape=jax.ShapeDtypeStruct((L, N // L), jnp.float32),
    )(conc16)


def kernel(state, edge_index, Wc, bc, W1, b1, W2, b2, W3, b3, deterministic=1):
    src = edge_index[0].reshape(NW, NCHUNK, CHUNK)
    dst = edge_index[1].reshape(NW, NCHUNK, CHUNK)

    degp = _deg_kernel(dst)
    g = _scale_call(state, Wc, degp)
    parts = _agg_kernel(g, src, dst)
    conc = _head_call(parts, g, state, degp, bc.reshape(1, D), W1,
                      b1.reshape(1, H), W2, b2.reshape(1, H),
                      W3.reshape(1, H), b3.reshape(1, 1))
    action = _norm_call(conc.reshape(L, N // L))
    return action.reshape(N // A, A)


# speedup vs baseline: 27.7459x; 1.0693x over previous
import jax
import jax.numpy as jnp
from jax import lax
from jax.experimental import pallas as pl
from jax.experimental.pallas import tpu as pltpu
from jax.experimental.pallas import tpu_sc as plsc

N = 10000
E = 320000
D = 128
H = 32
A = 8

NC = 2
NS = 16
NW = NC * NS
L = 16

EPT = E // NW
RPT = N // NS

DCH = 80
DNC = EPT // DCH

AC = 80
ANC = EPT // AC
RZC = RPT // AC
RZT = RPT % AC

ROWS_TC = 1000
GRID_TC = N // ROWS_TC

_mesh = plsc.VectorSubcoreMesh(core_axis_name="c", subcore_axis_name="s")
_sc_params = pltpu.CompilerParams(use_tc_tiling_on_sc=False)


def _deg_body(dst_hbm, out_hbm, dst_v, ones_v, zv, acc_sh):
    c = lax.axis_index("c")
    s = lax.axis_index("s")
    wid = c * NS + s
    base = s * RPT

    zeros16 = jnp.zeros((L,), jnp.float32)
    ones16 = jnp.ones((L,), jnp.float32)

    pltpu.sync_copy(dst_hbm.at[wid], dst_v)

    def _z(r, _):
        zv[r] = zeros16
        return 0
    lax.fori_loop(0, DCH, _z, 0)
    for t in range(RZC):
        pltpu.sync_copy(zv, acc_sh.at[pl.ds(base + t * AC, AC)])
    pltpu.sync_copy(zv.at[pl.ds(0, RZT)],
                    acc_sh.at[pl.ds(base + RPT - RZT, RZT)])

    def _o(r, _):
        ones_v[r] = ones16
        return 0
    lax.fori_loop(0, DCH, _o, 0)

    plsc.subcore_barrier()

    def _body(j, _):
        pltpu.sync_copy(ones_v, acc_sh.at[dst_v.at[pl.ds(j * DCH, DCH)]],
                        add=True)
        return 0
    lax.fori_loop(0, DNC, _body, 0)

    plsc.subcore_barrier()

    for t in range(RZC):
        pltpu.sync_copy(acc_sh.at[pl.ds(base + t * AC, AC)], zv)
        pltpu.sync_copy(zv, out_hbm.at[c, pl.ds(base + t * AC, AC)])
    pltpu.sync_copy(acc_sh.at[pl.ds(base + RPT - RZT, RZT)],
                    zv.at[pl.ds(0, RZT)])
    pltpu.sync_copy(zv.at[pl.ds(0, RZT)],
                    out_hbm.at[c, pl.ds(base + RPT - RZT, RZT)])


_deg_kernel = pl.kernel(
    _deg_body,
    out_type=jax.ShapeDtypeStruct((NC, N, L), jnp.float32),
    mesh=_mesh,
    scratch_types=[
        pltpu.VMEM((EPT,), jnp.int32),
        pltpu.VMEM((DCH, L), jnp.float32),
        pltpu.VMEM((AC, L), jnp.float32),
        pltpu.VMEM_SHARED((N, L), jnp.float32),
    ],
    compiler_params=_sc_params,
)


def _agg_body(g_hbm, src_hbm, dst_hbm, out_hbm, src_v, dst_v, buf_a, buf_b,
              acc_sh, sem_a, sem_b):
    c = lax.axis_index("c")
    s = lax.axis_index("s")
    wid = c * NS + s
    base = s * RPT

    zeros16 = jnp.zeros((L,), jnp.float32)

    pltpu.sync_copy(src_hbm.at[wid], src_v)
    pltpu.sync_copy(dst_hbm.at[wid], dst_v)

    def _z(r, _):
        for k in range(D // L):
            buf_a[r, pl.ds(k * L, L)] = zeros16
        return 0
    lax.fori_loop(0, AC, _z, 0)
    for t in range(RZC):
        pltpu.sync_copy(buf_a, acc_sh.at[pl.ds(base + t * AC, AC)])
    pltpu.sync_copy(buf_a.at[pl.ds(0, RZT)],
                    acc_sh.at[pl.ds(base + RPT - RZT, RZT)])

    plsc.subcore_barrier()

    pltpu.async_copy(g_hbm.at[src_v.at[pl.ds(0, AC)]], buf_a, sem_a)

    def _body(i, _):
        j0 = 2 * i
        j1 = 2 * i + 1
        pltpu.make_async_copy(
            g_hbm.at[src_v.at[pl.ds(j0 * AC, AC)]], buf_a, sem_a).wait()
        pltpu.async_copy(
            g_hbm.at[src_v.at[pl.ds(j1 * AC, AC)]], buf_b, sem_b)
        pltpu.sync_copy(buf_a, acc_sh.at[dst_v.at[pl.ds(j0 * AC, AC)]],
                        add=True)
        pltpu.make_async_copy(
            g_hbm.at[src_v.at[pl.ds(j1 * AC, AC)]], buf_b, sem_b).wait()

        @pl.when(j0 + 2 < ANC)
        def _():
            pltpu.async_copy(
                g_hbm.at[src_v.at[pl.ds((j0 + 2) * AC, AC)]], buf_a, sem_a)

        pltpu.sync_copy(buf_b, acc_sh.at[dst_v.at[pl.ds(j1 * AC, AC)]],
                        add=True)
        return 0
    lax.fori_loop(0, ANC // 2, _body, 0)

    pltpu.make_async_copy(
        g_hbm.at[src_v.at[pl.ds((ANC - 1) * AC, AC)]], buf_a, sem_a).wait()
    pltpu.sync_copy(buf_a, acc_sh.at[dst_v.at[pl.ds((ANC - 1) * AC, AC)]],
                    add=True)

    plsc.subcore_barrier()

    for t in range(RZC):
        pltpu.sync_copy(acc_sh.at[pl.ds(base + t * AC, AC)], buf_a)
        pltpu.sync_copy(buf_a, out_hbm.at[c, pl.ds(base + t * AC, AC)])
    pltpu.sync_copy(acc_sh.at[pl.ds(base + RPT - RZT, RZT)],
                    buf_a.at[pl.ds(0, RZT)])
    pltpu.sync_copy(buf_a.at[pl.ds(0, RZT)],
                    out_hbm.at[c, pl.ds(base + RPT - RZT, RZT)])


_agg_kernel = pl.kernel(
    _agg_body,
    out_type=jax.ShapeDtypeStruct((NC, N, D), jnp.float32),
    mesh=_mesh,
    scratch_types=[
        pltpu.VMEM((EPT,), jnp.int32),
        pltpu.VMEM((EPT,), jnp.int32),
        pltpu.VMEM((AC, D), jnp.float32),
        pltpu.VMEM((AC, D), jnp.float32),
        pltpu.VMEM_SHARED((N, D), jnp.float32),
        pltpu.SemaphoreType.DMA,
        pltpu.SemaphoreType.DMA,
    ],
    compiler_params=_sc_params,
)


def _scale_body(state_ref, wc_ref, degp_ref, g_ref):
    deg = degp_ref[0, :, 0] + degp_ref[1, :, 0] + 1.0
    dinv = lax.rsqrt(deg)
    h = jnp.dot(state_ref[...], wc_ref[...], preferred_element_type=jnp.float32)
    g_ref[...] = h * dinv[:, None]


def _scale_call(state, Wc, degp):
    return pl.pallas_call(
        _scale_body,
        grid=(GRID_TC,),
        in_specs=[
            pl.BlockSpec((ROWS_TC, D), lambda i: (i, 0)),
            pl.BlockSpec((D, D), lambda i: (0, 0)),
            pl.BlockSpec((NC, ROWS_TC, L), lambda i: (0, i, 0)),
        ],
        out_specs=pl.BlockSpec((ROWS_TC, D), lambda i: (i, 0)),
        out_shape=jax.ShapeDtypeStruct((N, D), jnp.float32),
    )(state, Wc, degp)


def _head_body(p_ref, g_ref, state_ref, degp_ref, bc_ref, w1_ref, b1_ref,
               w2_ref, b2_ref, w3_ref, b3_ref, conc_ref):
    deg = degp_ref[0, :, 0] + degp_ref[1, :, 0] + 1.0
    dinv = lax.rsqrt(deg)
    agg = p_ref[0] + p_ref[1] + g_ref[...]
    gcn = jnp.maximum(agg * dinv[:, None] + bc_ref[...], 0.0)
    x = gcn + state_ref[...]
    t = jnp.dot(x, w1_ref[...], preferred_element_type=jnp.float32) + b1_ref[...]
    t = jnp.where(t > 0, t, 0.01 * t)
    t = jnp.dot(t, w2_ref[...], preferred_element_type=jnp.float32) + b2_ref[...]
    t = jnp.where(t > 0, t, 0.01 * t)
    t = jnp.sum(t * w3_ref[...], axis=1, keepdims=True) + b3_ref[...]
    conc_ref[...] = jnp.maximum(t, 0.0) + jnp.log1p(jnp.exp(-jnp.abs(t)))


def _head_call(parts, g, state, degp, bc, W1, b1, W2, b2, W3r, b3):
    return pl.pallas_call(
        _head_body,
        grid=(GRID_TC,),
        in_specs=[
            pl.BlockSpec((NC, ROWS_TC, D), lambda i: (0, i, 0)),
            pl.BlockSpec((ROWS_TC, D), lambda i: (i, 0)),
            pl.BlockSpec((ROWS_TC, D), lambda i: (i, 0)),
            pl.BlockSpec((NC, ROWS_TC, L), lambda i: (0, i, 0)),
            pl.BlockSpec((1, D), lambda i: (0, 0)),
            pl.BlockSpec((D, H), lambda i: (0, 0)),
            pl.BlockSpec((1, H), lambda i: (0, 0)),
            pl.BlockSpec((H, H), lambda i: (0, 0)),
            pl.BlockSpec((1, H), lambda i: (0, 0)),
            pl.BlockSpec((1, H), lambda i: (0, 0)),
            pl.BlockSpec((1, 1), lambda i: (0, 0)),
        ],
        out_specs=pl.BlockSpec((ROWS_TC, 1), lambda i: (i, 0)),
        out_shape=jax.ShapeDtypeStruct((N, 1), jnp.float32),
    )(parts, g, state, degp, bc, W1, b1, W2, b2, W3r, b3)


def _norm_body(conc_ref, out_ref):
    cv = conc_ref[...]
    out_ref[...] = cv / (jnp.sum(cv) + 1e-20)


def _norm_call(conc16):
    return pl.pallas_call(
        _norm_body,
        out_shape=jax.ShapeDtypeStruct((L, N // L), jnp.float32),
    )(conc16)


def kernel(state, edge_index, Wc, bc, W1, b1, W2, b2, W3, b3, deterministic=1):
    src = edge_index[0].reshape(NW, EPT)
    dst = edge_index[1].reshape(NW, EPT)

    degp = _deg_kernel(dst)
    g = _scale_call(state, Wc, degp)
    parts = _agg_kernel(g, src, dst)
    conc = _head_call(parts, g, state, degp, bc.reshape(1, D), W1,
                      b1.reshape(1, H), W2, b2.reshape(1, H),
                      W3.reshape(1, H), b3.reshape(1, 1))
    action = _norm_call(conc.reshape(L, N // L))
    return action.reshape(N // A, A)
